# baseline (speedup 1.0000x reference)
import functools

import jax
import jax.numpy as jnp
from jax import lax
from jax.experimental import pallas as pl
from jax.experimental.pallas import tpu as pltpu
from jax.experimental.pallas import tpu_sc as plsc

f32 = jnp.float32
i32 = jnp.int32

_N = 10000
_NP = 10240
_E = 320000
_EP = 323584
_CH0 = 64
_CH1 = 64
_RPT = _NP // 16
_BLK = 1280
_NBLK = _NP // _BLK


def _tc_a_body(x_ref, win_ref, bin_ref, w0_ref, ac_ref, h_ref, s_ref):
    x0 = jnp.dot(x_ref[...], win_ref[...], preferred_element_type=f32)
    x0 = x0 + bin_ref[...]
    h = jnp.dot(x0, w0_ref[...], preferred_element_type=f32)
    h_ref[0] = h[:, :128]
    h_ref[1] = h[:, 128:]
    s_ref[...] = jnp.dot(h, ac_ref[...], preferred_element_type=f32)


def _tc_c_body(acc_ref, dt_ref, b_ref, y_ref, p_ref):
    i = pl.program_id(0)
    cols = []
    for c in range(2):
        a = acc_ref[c]
        for k in range(2):
            d = jnp.sum(dt_ref[c, k], axis=0)
            cols.append(a[:, 64 * k:64 * (k + 1)] / (d[:, None] + 1e-16))
    y = jnp.concatenate(cols, axis=1) + b_ref[...]
    y_ref[...] = y
    rows = i * _BLK + lax.broadcasted_iota(i32, (_BLK, 1), 0)
    ym = jnp.where(rows < _N, y, 0.0)
    p_ref[0, 0, :] = jnp.sum(ym, axis=0)
    p_ref[0, 1, :] = jnp.sum(ym * ym, axis=0)


def _tc_d_body(y_ref, st_ref, g_ref, be_ref, w1_ref, a1_ref, h_ref, s_ref):
    mu = st_ref[0:1, :]
    var = st_ref[1:2, :]
    xn = g_ref[...] * (y_ref[...] - mu) * lax.rsqrt(var + 1e-5) + be_ref[...]
    x1 = jnp.maximum(xn, 0.0)
    h = jnp.dot(x1, w1_ref[...], preferred_element_type=f32)
    h_ref[...] = h
    s_ref[...] = jnp.dot(h, a1_ref[...], preferred_element_type=f32)


def _tc_f_body(acc_ref, b_ref, y_ref, p_ref):
    i = pl.program_id(0)
    a = acc_ref[0] + acc_ref[1]
    y = a[:, 0:64] / (a[:, 64:65] + 1e-16) + b_ref[...]
    y_ref[...] = y
    rows = i * _BLK + lax.broadcasted_iota(i32, (_BLK, 1), 0)
    ym = jnp.where(rows < _N, y, 0.0)
    p_ref[0, 0, :] = jnp.sum(ym, axis=0)
    p_ref[0, 1, :] = jnp.sum(ym * ym, axis=0)


def _tc_g_body(y_ref, st_ref, g_ref, be_ref, wo1_ref, bo1_ref, wo2_ref,
               bo2_ref, o_ref):
    mu = st_ref[0:1, :]
    var = st_ref[1:2, :]
    x2 = g_ref[...] * (y_ref[...] - mu) * lax.rsqrt(var + 1e-5) + be_ref[...]
    h = jnp.maximum(jnp.dot(x2, wo1_ref[...], preferred_element_type=f32)
                    + bo1_ref[...], 0.0)
    o_ref[...] = jnp.dot(h, wo2_ref[...], preferred_element_type=f32) + bo2_ref[...]


def _full(shape):
    return pl.BlockSpec(shape, lambda i: tuple(0 for _ in shape))


def _sc_conv0_body(eids_h, hflat_h, s0tab_h, m_h, z_h, out_h, outd_h,
                   ids2, idxg, idxd, sbuf, eaa, eab, mv,
                   dta, dtb, gbuf, stage, acc, sem, sems):
    c = lax.axis_index("c")
    s = lax.axis_index("s")
    pltpu.sync_copy(m_h, mv)
    pltpu.sync_copy(z_h, acc.at[pl.ds(s * _RPT, _RPT)])

    def zrow(r, cc):
        sl = pl.ds(r * 16, 16)
        dta[sl] = jnp.zeros((16,), f32)
        dtb[sl] = jnp.zeros((16,), f32)
        return cc
    lax.fori_loop(0, _NP // 16, zrow, 0)
    plsc.subcore_barrier()

    m_a = plsc.load_gather(mv, [jnp.full((16,), 2 * c, i32)])
    m_b = plsc.load_gather(mv, [jnp.full((16,), 2 * c + 1, i32)])
    off = c * _NP
    nch = _EP // 16 // _CH0
    cbase = s * nch
    lane = lax.iota(i32, 16)
    lane0 = lane == 0
    zero16 = jnp.zeros((16,), i32)
    csa = jnp.full((16,), 2 * c, i32)
    csb = jnp.full((16,), 2 * c + 1, i32)
    cda = jnp.full((16,), 4 + 2 * c, i32)
    cdb = jnp.full((16,), 5 + 2 * c, i32)

    def load_ids(j, p):
        pltpu.sync_copy(eids_h.at[cbase + j], ids2.at[p])
        for g in range(4):
            sl = pl.ds(g * 16, 16)
            idxg[p, sl] = ids2[p, sl] + off
            idxd[p, sl] = ids2[p, pl.ds(64 + g * 16, 16)]

    load_ids(0, 0)
    pltpu.async_copy(hflat_h.at[idxg.at[0]], gbuf.at[0], sems.at[0])

    def chunk(j, carry):
        p = lax.rem(j, 2)
        q = 1 - p

        @pl.when(j + 1 < nch)
        def _prefetch():
            load_ids(j + 1, q)
            pltpu.async_copy(hflat_h.at[idxg.at[q]], gbuf.at[q], sems.at[q])

        pltpu.async_copy(s0tab_h.at[ids2.at[p]], sbuf, sem).wait()
        for g in range(_CH0 // 16):
            sl = pl.ds(g * 16, 16)
            rv = lane + g * 16
            rv2 = rv + 64
            sa = plsc.load_gather(sbuf, [rv, csa]) + plsc.load_gather(sbuf, [rv2, cda])
            sb = plsc.load_gather(sbuf, [rv, csb]) + plsc.load_gather(sbuf, [rv2, cdb])
            ea = jnp.where(sa >= 0, sa, 0.2 * sa) - m_a
            eb = jnp.where(sb >= 0, sb, 0.2 * sb) - m_b
            eaa[sl] = jnp.exp(ea)
            eab[sl] = jnp.exp(eb)
        pltpu.make_async_copy(hflat_h.at[idxg.at[p]], gbuf.at[p], sems.at[p]).wait()

        @plsc.parallel_loop(0, _CH0, step=1, unroll=4)
        def edge(e):
            ei = jnp.full((16,), e, i32)
            b_a = plsc.load_gather(eaa, [ei])
            b_b = plsc.load_gather(eab, [ei])
            vd = plsc.load_gather(idxd, [p * zero16 + p, ei])
            plsc.addupdate_scatter(dta, [vd], b_a, mask=lane0)
            plsc.addupdate_scatter(dtb, [vd], b_b, mask=lane0)
            for jj in range(8):
                sl = pl.ds(jj * 16, 16)
                stage[e, sl] = gbuf[p, e, sl] * (b_a if jj < 4 else b_b)
        pltpu.sync_copy(stage, acc.at[idxd.at[p]], add=True)
        return carry

    lax.fori_loop(0, nch, chunk, 0)
    plsc.subcore_barrier()
    pltpu.sync_copy(acc.at[pl.ds(s * _RPT, _RPT)],
                    out_h.at[c, pl.ds(s * _RPT, _RPT)])
    pltpu.sync_copy(dta, outd_h.at[c, 0, s])
    pltpu.sync_copy(dtb, outd_h.at[c, 1, s])


def _sc_conv1_body(eids_h, h1_h, s1tab_h, m_h, z_h, out_h,
                   ids2, idxd, sbuf, eaa, mv, gbuf, stage, acc, sem, sems):
    c = lax.axis_index("c")
    s = lax.axis_index("s")
    pltpu.sync_copy(m_h, mv)
    pltpu.sync_copy(z_h, acc.at[pl.ds(s * _RPT, _RPT)])

    def zrowst(e, cc):
        for jj in range(5, 8):
            stage[e, pl.ds(jj * 16, 16)] = jnp.zeros((16,), f32)
        return cc
    lax.fori_loop(0, _CH0, zrowst, 0)
    plsc.subcore_barrier()

    m_a = plsc.load_gather(mv, [jnp.full((16,), c, i32)])
    wid = s * 2 + c
    nch = _EP // 32 // _CH0
    cbase = wid * nch
    lane = lax.iota(i32, 16)
    zero16 = jnp.zeros((16,), i32)
    one16 = jnp.full((16,), 1, i32)

    def load_ids(j, p):
        pltpu.sync_copy(eids_h.at[cbase + j], ids2.at[p])
        for g in range(4):
            sl = pl.ds(g * 16, 16)
            idxd[p, sl] = ids2[p, pl.ds(64 + g * 16, 16)]

    load_ids(0, 0)
    pltpu.async_copy(h1_h.at[ids2.at[0, pl.ds(0, 64)]], gbuf.at[0], sems.at[0])

    def chunk(j, carry):
        p = lax.rem(j, 2)
        q = 1 - p

        @pl.when(j + 1 < nch)
        def _prefetch():
            load_ids(j + 1, q)
            pltpu.async_copy(h1_h.at[ids2.at[q, pl.ds(0, 64)]], gbuf.at[q],
                             sems.at[q])

        pltpu.async_copy(s1tab_h.at[ids2.at[p]], sbuf, sem).wait()
        for g in range(_CH0 // 16):
            sl = pl.ds(g * 16, 16)
            rv = lane + g * 16
            rv2 = rv + 64
            sa = plsc.load_gather(sbuf, [rv, zero16]) + plsc.load_gather(sbuf, [rv2, one16])
            ea = jnp.where(sa >= 0, sa, 0.2 * sa) - m_a
            eaa[sl] = jnp.exp(ea)
        pltpu.make_async_copy(h1_h.at[ids2.at[p, pl.ds(0, 64)]], gbuf.at[p],
                              sems.at[p]).wait()

        @plsc.parallel_loop(0, _CH0, step=1, unroll=4)
        def edge(e):
            b_a = plsc.load_gather(eaa, [jnp.full((16,), e, i32)])
            for jj in range(4):
                sl = pl.ds(jj * 16, 16)
                stage[e, sl] = gbuf[p, e, sl] * b_a
            tail = jnp.where(lane == 0, b_a, 0.0)
            stage[e, pl.ds(64, 16)] = tail
        pltpu.sync_copy(stage, acc.at[idxd.at[p]], add=True)
        return carry

    lax.fori_loop(0, nch, chunk, 0)
    plsc.subcore_barrier()
    pltpu.sync_copy(acc.at[pl.ds(s * _RPT, _RPT)],
                    out_h.at[c, pl.ds(s * _RPT, _RPT)])


@functools.cache
def _sc_kernels():
    mesh = plsc.VectorSubcoreMesh(core_axis_name="c", subcore_axis_name="s")
    params = pltpu.CompilerParams(needs_layout_passes=False,
                                  use_tc_tiling_on_sc=False)
    conv0 = pl.kernel(
        _sc_conv0_body,
        out_type=[
            jax.ShapeDtypeStruct((2, _NP, 128), f32),
            jax.ShapeDtypeStruct((2, 2, 16, _NP), f32),
        ],
        mesh=mesh,
        compiler_params=params,
        scratch_types=[
            pltpu.VMEM((2, 128), i32),
            pltpu.VMEM((2, _CH0), i32),
            pltpu.VMEM((2, _CH0), i32),
            pltpu.VMEM((128, 16), f32),
            pltpu.VMEM((_CH0,), f32),
            pltpu.VMEM((_CH0,), f32),
            pltpu.VMEM((128,), f32),
            pltpu.VMEM((_NP,), f32),
            pltpu.VMEM((_NP,), f32),
            pltpu.VMEM((2, _CH0, 128), f32),
            pltpu.VMEM((_CH0, 128), f32),
            pltpu.VMEM_SHARED((_NP, 128), f32),
            pltpu.SemaphoreType.DMA,
            pltpu.SemaphoreType.DMA((2,)),
        ],
    )
    conv1 = pl.kernel(
        _sc_conv1_body,
        out_type=jax.ShapeDtypeStruct((2, _NP, 128), f32),
        mesh=mesh,
        compiler_params=params,
        scratch_types=[
            pltpu.VMEM((2, 128), i32),
            pltpu.VMEM((2, _CH0), i32),
            pltpu.VMEM((128, 16), f32),
            pltpu.VMEM((_CH0,), f32),
            pltpu.VMEM((128,), f32),
            pltpu.VMEM((2, _CH0, 64), f32),
            pltpu.VMEM((_CH0, 128), f32),
            pltpu.VMEM_SHARED((_NP, 128), f32),
            pltpu.SemaphoreType.DMA,
            pltpu.SemaphoreType.DMA((2,)),
        ],
    )
    return conv0, conv1


def _leaky(x):
    return jnp.where(x >= 0, x, 0.2 * x)


def kernel(X, edge_index, W_in, b_in, W0, a_src0, a_dst0, bias0, g0, be0,
           W1, a_src1, a_dst1, bias1, g1, be1, Wo1, bo1, Wo2, bo2):
    x2 = X[:, :, -1].astype(f32)
    x2p = jnp.concatenate([x2, jnp.zeros((_NP - _N, 128), f32)], axis=0)
    src = edge_index[0].astype(i32)
    dst = edge_index[1].astype(i32)
    padi = jnp.full((_EP - _E,), _N, i32)
    srcp = jnp.concatenate([src, padi])
    dstp = jnp.concatenate([dst, padi])
    eids = jnp.concatenate([srcp.reshape(-1, _CH0), dstp.reshape(-1, _CH0)],
                           axis=1)

    eyeh = jnp.eye(4, dtype=f32)
    asrc = (a_src0[:, :, None] * eyeh[:, None, :]).reshape(256, 4)
    adst = (a_dst0[:, :, None] * eyeh[:, None, :]).reshape(256, 4)
    acomb = jnp.concatenate([asrc, adst, jnp.zeros((256, 120), f32)],
                            axis=1)
    a1c = jnp.concatenate([jnp.stack([a_src1[0], a_dst1[0]], axis=1),
                           jnp.zeros((64, 126), f32)], axis=1)

    colmask = jnp.arange(_NP) < _N

    h_split, s0t_raw = pl.pallas_call(
        _tc_a_body,
        grid=(_NBLK,),
        in_specs=[
            pl.BlockSpec((_BLK, 128), lambda i: (i, 0)),
            _full((128, 256)),
            _full((1, 256)),
            _full((256, 256)),
            _full((256, 128)),
        ],
        out_specs=[
            pl.BlockSpec((2, _BLK, 128), lambda i: (0, i, 0)),
            pl.BlockSpec((_BLK, 128), lambda i: (i, 0)),
        ],
        out_shape=[
            jax.ShapeDtypeStruct((2, _NP, 128), f32),
            jax.ShapeDtypeStruct((_NP, 128), f32),
        ],
    )(x2p, W_in, b_in[None], W0, acomb)

    s0t = jnp.where(colmask[:, None], s0t_raw[:, :8], -1e30)
    s0tab = jnp.concatenate([s0t, jnp.zeros((_NP, 8), f32)], axis=1)
    e0max = jnp.max(s0t[:, :4], axis=0) + jnp.max(s0t[:, 4:], axis=0)
    m0 = _leaky(e0max)
    mvec0 = jnp.concatenate([m0, jnp.zeros((124,), f32)])
    hflat = h_split.reshape(2 * _NP, 128)
    z0 = jnp.zeros((_RPT, 128), f32)

    sc_conv0, sc_conv1 = _sc_kernels()
    acc0, dt0 = sc_conv0(eids, hflat, s0tab, mvec0, z0)

    y0, p0 = pl.pallas_call(
        _tc_c_body,
        grid=(_NBLK,),
        in_specs=[
            pl.BlockSpec((2, _BLK, 128), lambda i: (0, i, 0)),
            pl.BlockSpec((2, 2, 16, _BLK), lambda i: (0, 0, 0, i)),
            _full((1, 256)),
        ],
        out_specs=[
            pl.BlockSpec((_BLK, 256), lambda i: (i, 0)),
            pl.BlockSpec((1, 2, 256), lambda i: (i, 0, 0)),
        ],
        out_shape=[
            jax.ShapeDtypeStruct((_NP, 256), f32),
            jax.ShapeDtypeStruct((_NBLK, 2, 256), f32),
        ],
    )(acc0, dt0, bias0[None])

    mu0 = jnp.sum(p0[:, 0, :], axis=0) / _N
    var0 = jnp.sum(p0[:, 1, :], axis=0) / _N - mu0 * mu0
    st0 = jnp.stack([mu0, var0])

    h1, s1t_raw = pl.pallas_call(
        _tc_d_body,
        grid=(_NBLK,),
        in_specs=[
            pl.BlockSpec((_BLK, 256), lambda i: (i, 0)),
            _full((2, 256)),
            _full((1, 256)),
            _full((1, 256)),
            _full((256, 64)),
            _full((64, 128)),
        ],
        out_specs=[
            pl.BlockSpec((_BLK, 64), lambda i: (i, 0)),
            pl.BlockSpec((_BLK, 128), lambda i: (i, 0)),
        ],
        out_shape=[
            jax.ShapeDtypeStruct((_NP, 64), f32),
            jax.ShapeDtypeStruct((_NP, 128), f32),
        ],
    )(y0, st0, g0[None], be0[None], W1, a1c)

    s1t = jnp.where(colmask[:, None], s1t_raw[:, :2], -1e30)
    s1tab = jnp.concatenate([s1t, jnp.zeros((_NP, 14), f32)], axis=1)
    m1 = _leaky(jnp.max(s1t[:, 0]) + jnp.max(s1t[:, 1]))
    mvec1 = jnp.concatenate([m1[None], m1[None], jnp.zeros((126,), f32)])

    acc1 = sc_conv1(eids, h1, s1tab, mvec1, z0)

    y1, p1 = pl.pallas_call(
        _tc_f_body,
        grid=(_NBLK,),
        in_specs=[
            pl.BlockSpec((2, _BLK, 128), lambda i: (0, i, 0)),
            _full((1, 64)),
        ],
        out_specs=[
            pl.BlockSpec((_BLK, 64), lambda i: (i, 0)),
            pl.BlockSpec((1, 2, 64), lambda i: (i, 0, 0)),
        ],
        out_shape=[
            jax.ShapeDtypeStruct((_NP, 64), f32),
            jax.ShapeDtypeStruct((_NBLK, 2, 64), f32),
        ],
    )(acc1, bias1[None])

    mu1 = jnp.sum(p1[:, 0, :], axis=0) / _N
    var1 = jnp.sum(p1[:, 1, :], axis=0) / _N - mu1 * mu1
    st1 = jnp.stack([mu1, var1])

    outp = pl.pallas_call(
        _tc_g_body,
        grid=(_NBLK,),
        in_specs=[
            pl.BlockSpec((_BLK, 64), lambda i: (i, 0)),
            _full((2, 64)),
            _full((1, 64)),
            _full((1, 64)),
            _full((64, 128)),
            _full((1, 128)),
            _full((128, 1)),
            _full((1, 1)),
        ],
        out_specs=pl.BlockSpec((_BLK, 1), lambda i: (i, 0)),
        out_shape=jax.ShapeDtypeStruct((_NP, 1), f32),
    )(y1, st1, g1[None], be1[None], Wo1, bo1[None], Wo2, bo2[None])

    return outp[:_N]


# speedup vs baseline: 1.2099x; 1.2099x over previous
import functools

import jax
import jax.numpy as jnp
from jax import lax
from jax.experimental import pallas as pl
from jax.experimental.pallas import tpu as pltpu
from jax.experimental.pallas import tpu_sc as plsc

f32 = jnp.float32
i32 = jnp.int32

_N = 10000
_NP = 10240
_E = 320000
_EP = 323584
_CH0 = 64
_CH1 = 64
_RPT = _NP // 16
_BLK = 1280
_NBLK = _NP // _BLK


def _tc_a_body(x_ref, win_ref, bin_ref, w0_ref, ac_ref, h_ref, s_ref):
    x0 = jnp.dot(x_ref[...], win_ref[...], preferred_element_type=f32)
    x0 = x0 + bin_ref[...]
    h = jnp.dot(x0, w0_ref[...], preferred_element_type=f32)
    h_ref[0] = h[:, :128]
    h_ref[1] = h[:, 128:]
    s_ref[...] = jnp.dot(h, ac_ref[...], preferred_element_type=f32)


def _tc_c_body(acc_ref, dt_ref, b_ref, y_ref, p_ref):
    i = pl.program_id(0)
    cols = []
    for c in range(2):
        a = acc_ref[c]
        for k in range(2):
            d = jnp.sum(dt_ref[c, k], axis=0)
            cols.append(a[:, 64 * k:64 * (k + 1)] / (d[:, None] + 1e-16))
    y = jnp.concatenate(cols, axis=1) + b_ref[...]
    y_ref[...] = y
    rows = i * _BLK + lax.broadcasted_iota(i32, (_BLK, 1), 0)
    ym = jnp.where(rows < _N, y, 0.0)
    p_ref[0, 0, :] = jnp.sum(ym, axis=0)
    p_ref[0, 1, :] = jnp.sum(ym * ym, axis=0)


def _tc_d_body(y_ref, st_ref, g_ref, be_ref, w1_ref, a1_ref, h_ref, s_ref):
    mu = st_ref[0:1, :]
    var = st_ref[1:2, :]
    xn = g_ref[...] * (y_ref[...] - mu) * lax.rsqrt(var + 1e-5) + be_ref[...]
    x1 = jnp.maximum(xn, 0.0)
    h = jnp.dot(x1, w1_ref[...], preferred_element_type=f32)
    h_ref[...] = h
    s_ref[...] = jnp.dot(h, a1_ref[...], preferred_element_type=f32)


def _tc_f_body(acc_ref, b_ref, y_ref, p_ref):
    i = pl.program_id(0)
    a = acc_ref[0] + acc_ref[1]
    y = a[:, 0:64] / (a[:, 64:65] + 1e-16) + b_ref[...]
    y_ref[...] = y
    rows = i * _BLK + lax.broadcasted_iota(i32, (_BLK, 1), 0)
    ym = jnp.where(rows < _N, y, 0.0)
    p_ref[0, 0, :] = jnp.sum(ym, axis=0)
    p_ref[0, 1, :] = jnp.sum(ym * ym, axis=0)


def _tc_g_body(y_ref, st_ref, g_ref, be_ref, wo1_ref, bo1_ref, wo2_ref,
               bo2_ref, o_ref):
    mu = st_ref[0:1, :]
    var = st_ref[1:2, :]
    x2 = g_ref[...] * (y_ref[...] - mu) * lax.rsqrt(var + 1e-5) + be_ref[...]
    h = jnp.maximum(jnp.dot(x2, wo1_ref[...], preferred_element_type=f32)
                    + bo1_ref[...], 0.0)
    o_ref[...] = jnp.dot(h, wo2_ref[...], preferred_element_type=f32) + bo2_ref[...]


def _full(shape):
    return pl.BlockSpec(shape, lambda i: tuple(0 for _ in shape))


def _sc_conv0_body(eids_h, hflat_h, s0tab_h, m_h, z_h, out_h, outd_h,
                   ids2, idxg, idxd, sbuf, eaa, eab, mv,
                   dta, dtb, gbuf, stage, acc, sem, sems):
    c = lax.axis_index("c")
    s = lax.axis_index("s")
    pltpu.sync_copy(m_h, mv)
    pltpu.sync_copy(z_h, acc.at[pl.ds(s * _RPT, _RPT)])

    def zrow(r, cc):
        sl = pl.ds(r * 16, 16)
        dta[sl] = jnp.zeros((16,), f32)
        dtb[sl] = jnp.zeros((16,), f32)
        return cc
    lax.fori_loop(0, _NP // 16, zrow, 0)
    plsc.subcore_barrier()

    m_a = plsc.load_gather(mv, [jnp.full((16,), 2 * c, i32)])
    m_b = plsc.load_gather(mv, [jnp.full((16,), 2 * c + 1, i32)])
    off = c * _NP
    nch = _EP // 16 // _CH0
    cbase = s * nch
    lane = lax.iota(i32, 16)
    lane0 = lane == 0
    zero16 = jnp.zeros((16,), i32)
    csa = jnp.full((16,), 2 * c, i32)
    csb = jnp.full((16,), 2 * c + 1, i32)
    cda = jnp.full((16,), 4 + 2 * c, i32)
    cdb = jnp.full((16,), 5 + 2 * c, i32)

    def load_ids(j, p):
        pltpu.sync_copy(eids_h.at[cbase + j], ids2.at[p])
        for g in range(4):
            sl = pl.ds(g * 16, 16)
            idxg[p, sl] = ids2[p, sl] + off
            idxd[p, sl] = ids2[p, pl.ds(64 + g * 16, 16)]

    load_ids(0, 0)
    pltpu.async_copy(hflat_h.at[idxg.at[0]], gbuf.at[0], sems.at[0])

    def chunk(j, carry):
        p = lax.rem(j, 2)
        q = 1 - p

        sdesc = pltpu.async_copy(s0tab_h.at[ids2.at[p]], sbuf, sem)

        @pl.when(j + 1 < nch)
        def _prefetch():
            load_ids(j + 1, q)
            pltpu.async_copy(hflat_h.at[idxg.at[q]], gbuf.at[q], sems.at[q])

        sdesc.wait()
        for g in range(_CH0 // 16):
            sl = pl.ds(g * 16, 16)
            rv = lane + g * 16
            rv2 = rv + 64
            sa = plsc.load_gather(sbuf, [rv, csa]) + plsc.load_gather(sbuf, [rv2, cda])
            sb = plsc.load_gather(sbuf, [rv, csb]) + plsc.load_gather(sbuf, [rv2, cdb])
            ea = jnp.where(sa >= 0, sa, 0.2 * sa) - m_a
            eb = jnp.where(sb >= 0, sb, 0.2 * sb) - m_b
            eaa[sl] = jnp.exp(ea)
            eab[sl] = jnp.exp(eb)
        pltpu.make_async_copy(hflat_h.at[idxg.at[p]], gbuf.at[p], sems.at[p]).wait()

        @plsc.parallel_loop(0, _CH0, step=1, unroll=4)
        def edge(e):
            ei = jnp.full((16,), e, i32)
            b_a = plsc.load_gather(eaa, [ei])
            b_b = plsc.load_gather(eab, [ei])
            vd = plsc.load_gather(idxd, [p * zero16 + p, ei])
            plsc.addupdate_scatter(dta, [vd], b_a, mask=lane0)
            plsc.addupdate_scatter(dtb, [vd], b_b, mask=lane0)
            for jj in range(8):
                sl = pl.ds(jj * 16, 16)
                stage[e, sl] = gbuf[p, e, sl] * (b_a if jj < 4 else b_b)
        pltpu.sync_copy(stage, acc.at[idxd.at[p]], add=True)
        return carry

    lax.fori_loop(0, nch, chunk, 0)
    plsc.subcore_barrier()
    pltpu.sync_copy(acc.at[pl.ds(s * _RPT, _RPT)],
                    out_h.at[c, pl.ds(s * _RPT, _RPT)])
    pltpu.sync_copy(dta, outd_h.at[c, 0, s])
    pltpu.sync_copy(dtb, outd_h.at[c, 1, s])


def _sc_conv1_body(eids_h, h1_h, s1tab_h, m_h, z_h, out_h,
                   ids2, idxd, sbuf, eaa, mv, gbuf, stage, acc, sem, sems):
    c = lax.axis_index("c")
    s = lax.axis_index("s")
    pltpu.sync_copy(m_h, mv)
    pltpu.sync_copy(z_h, acc.at[pl.ds(s * _RPT, _RPT)])

    def zrowst(e, cc):
        for jj in range(5, 8):
            stage[e, pl.ds(jj * 16, 16)] = jnp.zeros((16,), f32)
        return cc
    lax.fori_loop(0, _CH0, zrowst, 0)
    plsc.subcore_barrier()

    m_a = plsc.load_gather(mv, [jnp.full((16,), c, i32)])
    wid = s * 2 + c
    nch = _EP // 32 // _CH0
    cbase = wid * nch
    lane = lax.iota(i32, 16)
    zero16 = jnp.zeros((16,), i32)
    one16 = jnp.full((16,), 1, i32)

    def load_ids(j, p):
        pltpu.sync_copy(eids_h.at[cbase + j], ids2.at[p])
        for g in range(4):
            sl = pl.ds(g * 16, 16)
            idxd[p, sl] = ids2[p, pl.ds(64 + g * 16, 16)]

    load_ids(0, 0)
    pltpu.async_copy(h1_h.at[ids2.at[0, pl.ds(0, 64)]], gbuf.at[0], sems.at[0])

    def chunk(j, carry):
        p = lax.rem(j, 2)
        q = 1 - p

        sdesc = pltpu.async_copy(s1tab_h.at[ids2.at[p]], sbuf, sem)

        @pl.when(j + 1 < nch)
        def _prefetch():
            load_ids(j + 1, q)
            pltpu.async_copy(h1_h.at[ids2.at[q, pl.ds(0, 64)]], gbuf.at[q],
                             sems.at[q])

        sdesc.wait()
        for g in range(_CH0 // 16):
            sl = pl.ds(g * 16, 16)
            rv = lane + g * 16
            rv2 = rv + 64
            sa = plsc.load_gather(sbuf, [rv, zero16]) + plsc.load_gather(sbuf, [rv2, one16])
            ea = jnp.where(sa >= 0, sa, 0.2 * sa) - m_a
            eaa[sl] = jnp.exp(ea)
        pltpu.make_async_copy(h1_h.at[ids2.at[p, pl.ds(0, 64)]], gbuf.at[p],
                              sems.at[p]).wait()

        @plsc.parallel_loop(0, _CH0, step=1, unroll=4)
        def edge(e):
            b_a = plsc.load_gather(eaa, [jnp.full((16,), e, i32)])
            for jj in range(4):
                sl = pl.ds(jj * 16, 16)
                stage[e, sl] = gbuf[p, e, sl] * b_a
            tail = jnp.where(lane == 0, b_a, 0.0)
            stage[e, pl.ds(64, 16)] = tail
        pltpu.sync_copy(stage, acc.at[idxd.at[p]], add=True)
        return carry

    lax.fori_loop(0, nch, chunk, 0)
    plsc.subcore_barrier()
    pltpu.sync_copy(acc.at[pl.ds(s * _RPT, _RPT)],
                    out_h.at[c, pl.ds(s * _RPT, _RPT)])


@functools.cache
def _sc_kernels():
    mesh = plsc.VectorSubcoreMesh(core_axis_name="c", subcore_axis_name="s")
    params = pltpu.CompilerParams(needs_layout_passes=False,
                                  use_tc_tiling_on_sc=False)
    conv0 = pl.kernel(
        _sc_conv0_body,
        out_type=[
            jax.ShapeDtypeStruct((2, _NP, 128), f32),
            jax.ShapeDtypeStruct((2, 2, 16, _NP), f32),
        ],
        mesh=mesh,
        compiler_params=params,
        scratch_types=[
            pltpu.VMEM((2, 128), i32),
            pltpu.VMEM((2, _CH0), i32),
            pltpu.VMEM((2, _CH0), i32),
            pltpu.VMEM((128, 16), f32),
            pltpu.VMEM((_CH0,), f32),
            pltpu.VMEM((_CH0,), f32),
            pltpu.VMEM((128,), f32),
            pltpu.VMEM((_NP,), f32),
            pltpu.VMEM((_NP,), f32),
            pltpu.VMEM((2, _CH0, 128), f32),
            pltpu.VMEM((_CH0, 128), f32),
            pltpu.VMEM_SHARED((_NP, 128), f32),
            pltpu.SemaphoreType.DMA,
            pltpu.SemaphoreType.DMA((2,)),
        ],
    )
    conv1 = pl.kernel(
        _sc_conv1_body,
        out_type=jax.ShapeDtypeStruct((2, _NP, 128), f32),
        mesh=mesh,
        compiler_params=params,
        scratch_types=[
            pltpu.VMEM((2, 128), i32),
            pltpu.VMEM((2, _CH0), i32),
            pltpu.VMEM((128, 16), f32),
            pltpu.VMEM((_CH0,), f32),
            pltpu.VMEM((128,), f32),
            pltpu.VMEM((2, _CH0, 64), f32),
            pltpu.VMEM((_CH0, 128), f32),
            pltpu.VMEM_SHARED((_NP, 128), f32),
            pltpu.SemaphoreType.DMA,
            pltpu.SemaphoreType.DMA((2,)),
        ],
    )
    return conv0, conv1


def _leaky(x):
    return jnp.where(x >= 0, x, 0.2 * x)


def kernel(X, edge_index, W_in, b_in, W0, a_src0, a_dst0, bias0, g0, be0,
           W1, a_src1, a_dst1, bias1, g1, be1, Wo1, bo1, Wo2, bo2):
    x2 = X[:, :, -1].astype(f32)
    x2p = jnp.concatenate([x2, jnp.zeros((_NP - _N, 128), f32)], axis=0)
    src = edge_index[0].astype(i32)
    dst = edge_index[1].astype(i32)
    padi = jnp.full((_EP - _E,), _N, i32)
    srcp = jnp.concatenate([src, padi])
    dstp = jnp.concatenate([dst, padi])
    eids = jnp.concatenate([srcp.reshape(-1, _CH0), dstp.reshape(-1, _CH0)],
                           axis=1)

    eyeh = jnp.eye(4, dtype=f32)
    asrc = (a_src0[:, :, None] * eyeh[:, None, :]).reshape(256, 4)
    adst = (a_dst0[:, :, None] * eyeh[:, None, :]).reshape(256, 4)
    acomb = jnp.concatenate([asrc, adst, jnp.zeros((256, 120), f32)],
                            axis=1)
    a1c = jnp.concatenate([jnp.stack([a_src1[0], a_dst1[0]], axis=1),
                           jnp.zeros((64, 126), f32)], axis=1)

    colmask = jnp.arange(_NP) < _N

    h_split, s0t_raw = pl.pallas_call(
        _tc_a_body,
        grid=(_NBLK,),
        in_specs=[
            pl.BlockSpec((_BLK, 128), lambda i: (i, 0)),
            _full((128, 256)),
            _full((1, 256)),
            _full((256, 256)),
            _full((256, 128)),
        ],
        out_specs=[
            pl.BlockSpec((2, _BLK, 128), lambda i: (0, i, 0)),
            pl.BlockSpec((_BLK, 128), lambda i: (i, 0)),
        ],
        out_shape=[
            jax.ShapeDtypeStruct((2, _NP, 128), f32),
            jax.ShapeDtypeStruct((_NP, 128), f32),
        ],
    )(x2p, W_in, b_in[None], W0, acomb)

    s0t = jnp.where(colmask[:, None], s0t_raw[:, :8], -1e30)
    s0tab = jnp.concatenate([s0t, jnp.zeros((_NP, 8), f32)], axis=1)
    e0max = jnp.max(s0t[:, :4], axis=0) + jnp.max(s0t[:, 4:], axis=0)
    m0 = _leaky(e0max)
    mvec0 = jnp.concatenate([m0, jnp.zeros((124,), f32)])
    hflat = h_split.reshape(2 * _NP, 128)
    z0 = jnp.zeros((_RPT, 128), f32)

    sc_conv0, sc_conv1 = _sc_kernels()
    acc0, dt0 = sc_conv0(eids, hflat, s0tab, mvec0, z0)

    y0, p0 = pl.pallas_call(
        _tc_c_body,
        grid=(_NBLK,),
        in_specs=[
            pl.BlockSpec((2, _BLK, 128), lambda i: (0, i, 0)),
            pl.BlockSpec((2, 2, 16, _BLK), lambda i: (0, 0, 0, i)),
            _full((1, 256)),
        ],
        out_specs=[
            pl.BlockSpec((_BLK, 256), lambda i: (i, 0)),
            pl.BlockSpec((1, 2, 256), lambda i: (i, 0, 0)),
        ],
        out_shape=[
            jax.ShapeDtypeStruct((_NP, 256), f32),
            jax.ShapeDtypeStruct((_NBLK, 2, 256), f32),
        ],
    )(acc0, dt0, bias0[None])

    mu0 = jnp.sum(p0[:, 0, :], axis=0) / _N
    var0 = jnp.sum(p0[:, 1, :], axis=0) / _N - mu0 * mu0
    st0 = jnp.stack([mu0, var0])

    h1, s1t_raw = pl.pallas_call(
        _tc_d_body,
        grid=(_NBLK,),
        in_specs=[
            pl.BlockSpec((_BLK, 256), lambda i: (i, 0)),
            _full((2, 256)),
            _full((1, 256)),
            _full((1, 256)),
            _full((256, 64)),
            _full((64, 128)),
        ],
        out_specs=[
            pl.BlockSpec((_BLK, 64), lambda i: (i, 0)),
            pl.BlockSpec((_BLK, 128), lambda i: (i, 0)),
        ],
        out_shape=[
            jax.ShapeDtypeStruct((_NP, 64), f32),
            jax.ShapeDtypeStruct((_NP, 128), f32),
        ],
    )(y0, st0, g0[None], be0[None], W1, a1c)

    s1t = jnp.where(colmask[:, None], s1t_raw[:, :2], -1e30)
    s1tab = jnp.concatenate([s1t, jnp.zeros((_NP, 14), f32)], axis=1)
    m1 = _leaky(jnp.max(s1t[:, 0]) + jnp.max(s1t[:, 1]))
    mvec1 = jnp.concatenate([m1[None], m1[None], jnp.zeros((126,), f32)])

    acc1 = sc_conv1(eids, h1, s1tab, mvec1, z0)

    y1, p1 = pl.pallas_call(
        _tc_f_body,
        grid=(_NBLK,),
        in_specs=[
            pl.BlockSpec((2, _BLK, 128), lambda i: (0, i, 0)),
            _full((1, 64)),
        ],
        out_specs=[
            pl.BlockSpec((_BLK, 64), lambda i: (i, 0)),
            pl.BlockSpec((1, 2, 64), lambda i: (i, 0, 0)),
        ],
        out_shape=[
            jax.ShapeDtypeStruct((_NP, 64), f32),
            jax.ShapeDtypeStruct((_NBLK, 2, 64), f32),
        ],
    )(acc1, bias1[None])

    mu1 = jnp.sum(p1[:, 0, :], axis=0) / _N
    var1 = jnp.sum(p1[:, 1, :], axis=0) / _N - mu1 * mu1
    st1 = jnp.stack([mu1, var1])

    outp = pl.pallas_call(
        _tc_g_body,
        grid=(_NBLK,),
        in_specs=[
            pl.BlockSpec((_BLK, 64), lambda i: (i, 0)),
            _full((2, 64)),
            _full((1, 64)),
            _full((1, 64)),
            _full((64, 128)),
            _full((1, 128)),
            _full((128, 1)),
            _full((1, 1)),
        ],
        out_specs=pl.BlockSpec((_BLK, 1), lambda i: (i, 0)),
        out_shape=jax.ShapeDtypeStruct((_NP, 1), f32),
    )(y1, st1, g1[None], be1[None], Wo1, bo1[None], Wo2, bo2[None])

    return outp[:_N]
